# baseline (speedup 1.0000x reference)
import functools
import jax
import jax.numpy as jnp
from jax import lax
from jax.experimental import pallas as pl
from jax.experimental.pallas import tpu as pltpu
from jax.experimental.pallas import tpu_sc as plsc

EMB = 16
NC = 2
NS = 16
NW = NC * NS
EB = 2560
CH = 2000

_MESH = functools.partial(
    plsc.VectorSubcoreMesh, core_axis_name="c", subcore_axis_name="s",
    num_cores=NC, num_subcores=NS)
_SC_PARAMS = pltpu.CompilerParams(use_tc_tiling_on_sc=False)


def _sc_gather(h, src):
    M = src.shape[0]
    per_w = M // NW
    n_chunks = per_w // CH

    @functools.partial(
        pl.kernel,
        out_type=jax.ShapeDtypeStruct((M, EMB), jnp.float32),
        mesh=_MESH(),
        scratch_types=[
            pltpu.VMEM((CH,), jnp.int32),
            pltpu.VMEM((CH, EMB), jnp.float32),
            pltpu.SemaphoreType.DMA,
        ],
        compiler_params=_SC_PARAMS,
    )
    def k(h_hbm, src_hbm, out_hbm, idx_v, rows_v, sem):
        wid = lax.axis_index("s") * NC + lax.axis_index("c")
        base = wid * per_w
        for c in range(n_chunks):
            off = base + c * CH
            pltpu.sync_copy(src_hbm.at[pl.ds(off, CH)], idx_v)
            pltpu.async_copy(h_hbm.at[idx_v], rows_v, sem).wait()
            pltpu.sync_copy(rows_v, out_hbm.at[pl.ds(off, CH)])

    return k(h, src)


def _sc_scatter(m, dst, n_nodes):
    M = m.shape[0]
    per_w = M // NW
    n_chunks = per_w // CH
    rpt = n_nodes // NS

    @functools.partial(
        pl.kernel,
        out_type=jax.ShapeDtypeStruct((NC, n_nodes, EMB), jnp.float32),
        mesh=_MESH(),
        scratch_types=[
            pltpu.VMEM((CH,), jnp.int32),
            pltpu.VMEM((CH, EMB), jnp.float32),
            pltpu.VMEM_SHARED((n_nodes, EMB), jnp.float32),
            pltpu.SemaphoreType.DMA,
        ],
        compiler_params=_SC_PARAMS,
    )
    def k(m_hbm, dst_hbm, out_hbm, idx_v, rows_v, acc_sh, sem):
        cid = lax.axis_index("c")
        sid = lax.axis_index("s")
        wid = sid * NC + cid

        def zero_row(i, _):
            rows_v[i, :] = jnp.zeros((EMB,), jnp.float32)
            return 0
        lax.fori_loop(0, rpt, zero_row, 0)
        pltpu.sync_copy(rows_v.at[pl.ds(0, rpt)], acc_sh.at[pl.ds(sid * rpt, rpt)])
        plsc.subcore_barrier()

        base = wid * per_w
        for c in range(n_chunks):
            off = base + c * CH
            pltpu.sync_copy(dst_hbm.at[pl.ds(off, CH)], idx_v)
            pltpu.sync_copy(m_hbm.at[pl.ds(off, CH)], rows_v)
            pltpu.sync_copy(rows_v, acc_sh.at[idx_v], add=True)
        plsc.subcore_barrier()

        pltpu.sync_copy(acc_sh.at[pl.ds(sid * rpt, rpt)], rows_v.at[pl.ds(0, rpt)])
        pltpu.sync_copy(rows_v.at[pl.ds(0, rpt)], out_hbm.at[cid, pl.ds(sid * rpt, rpt)])

    return k(m, dst)


def _dot(x, y):
    return jax.lax.dot_general(x, y, (((1,), (0,)), ((), ())),
                               preferred_element_type=jnp.float32)


RB = EB // 8
PK = 8 * EMB
ZW = 8 * EMB * EMB


def _msg_body(a_ref, hj_ref, r_ref, t_ref, w2_ref, bt_ref, out_ref):
    a = a_ref[...].astype(jnp.bfloat16)
    hj = hj_ref[...].astype(jnp.bfloat16)
    z = _dot(a, r_ref[...]) * _dot(hj, t_ref[...])
    m = _dot(z.astype(jnp.bfloat16), w2_ref[...]) + _dot(hj, bt_ref[...])
    out_ref[...] = m


def _messages(a_p, hj_p, W2p, Btp, Rp, Tp):
    Mr = a_p.shape[0]
    return pl.pallas_call(
        _msg_body,
        grid=(Mr // RB,),
        in_specs=[
            pl.BlockSpec((RB, PK), lambda i: (i, 0)),
            pl.BlockSpec((RB, PK), lambda i: (i, 0)),
            pl.BlockSpec((PK, ZW), lambda i: (0, 0)),
            pl.BlockSpec((PK, ZW), lambda i: (0, 0)),
            pl.BlockSpec((ZW, PK), lambda i: (0, 0)),
            pl.BlockSpec((PK, PK), lambda i: (0, 0)),
        ],
        out_specs=pl.BlockSpec((RB, PK), lambda i: (i, 0)),
        out_shape=jax.ShapeDtypeStruct((Mr, PK), jnp.float32),
    )(a_p, hj_p, Rp, Tp, W2p, Btp)


def _ln_body(h_ref, p_ref, g_ref, b_ref, out_ref):
    x = h_ref[...] + p_ref[0] + p_ref[1]
    mean = jnp.mean(x, axis=-1, keepdims=True)
    var = jnp.mean((x - mean) ** 2, axis=-1, keepdims=True)
    out_ref[...] = (x - mean) * jax.lax.rsqrt(var + 1e-5) * g_ref[...] + b_ref[...]


def _layer_norm(h, partials, gamma, beta):
    N = h.shape[0]
    NB = 2000
    return pl.pallas_call(
        _ln_body,
        grid=(N // NB,),
        in_specs=[
            pl.BlockSpec((NB, EMB), lambda i: (i, 0)),
            pl.BlockSpec((NC, NB, EMB), lambda i: (0, i, 0)),
            pl.BlockSpec((1, EMB), lambda i: (0, 0)),
            pl.BlockSpec((1, EMB), lambda i: (0, 0)),
        ],
        out_specs=pl.BlockSpec((NB, EMB), lambda i: (i, 0)),
        out_shape=jax.ShapeDtypeStruct((N, EMB), jnp.float32),
    )(h, partials, gamma.reshape(1, EMB), beta.reshape(1, EMB))


def kernel(h, edge_index, edge_attr, W_lin, b_lin, ln_gamma, ln_beta):
    dst = edge_index[0].astype(jnp.int32)
    src = edge_index[1].astype(jnp.int32)
    M = edge_attr.shape[0]
    W2 = W_lin.reshape(EMB, EMB, EMB).transpose(2, 1, 0).reshape(EMB * EMB, EMB)
    Bt = b_lin.reshape(EMB, EMB).T
    eye = jnp.eye(EMB, dtype=jnp.float32)
    R = jnp.repeat(eye, EMB, axis=1)
    T = jnp.tile(eye, (1, EMB))
    eye8 = jnp.eye(8, dtype=jnp.float32)
    Rp = jnp.kron(eye8, R).astype(jnp.bfloat16)
    Tp = jnp.kron(eye8, T).astype(jnp.bfloat16)
    W2p = jnp.kron(eye8, W2).astype(jnp.bfloat16)
    Btp = jnp.kron(eye8, Bt).astype(jnp.bfloat16)

    h_j = _sc_gather(h, src)
    m = _messages(edge_attr.reshape(M // 8, PK), h_j.reshape(M // 8, PK),
                  W2p, Btp, Rp, Tp)
    partials = _sc_scatter(m.reshape(M, EMB), dst, h.shape[0])
    return _layer_norm(h, partials, ln_gamma, ln_beta)


# speedup vs baseline: 1.0531x; 1.0531x over previous
import functools
import jax
import jax.numpy as jnp
from jax import lax
from jax.experimental import pallas as pl
from jax.experimental.pallas import tpu as pltpu
from jax.experimental.pallas import tpu_sc as plsc

EMB = 16
NC = 2
NS = 16
NW = NC * NS
EB = 16000
CH = 2000

_MESH = functools.partial(
    plsc.VectorSubcoreMesh, core_axis_name="c", subcore_axis_name="s",
    num_cores=NC, num_subcores=NS)
_SC_PARAMS = pltpu.CompilerParams(use_tc_tiling_on_sc=False)


def _sc_gather(h, src):
    M = src.shape[0]
    per_w = M // NW
    n_chunks = per_w // CH

    @functools.partial(
        pl.kernel,
        out_type=jax.ShapeDtypeStruct((M, EMB), jnp.float32),
        mesh=_MESH(),
        scratch_types=[
            pltpu.VMEM((CH,), jnp.int32),
            pltpu.VMEM((CH, EMB), jnp.float32),
            pltpu.SemaphoreType.DMA,
        ],
        compiler_params=_SC_PARAMS,
    )
    def k(h_hbm, src_hbm, out_hbm, idx_v, rows_v, sem):
        wid = lax.axis_index("s") * NC + lax.axis_index("c")
        base = wid * per_w
        for c in range(n_chunks):
            off = base + c * CH
            pltpu.sync_copy(src_hbm.at[pl.ds(off, CH)], idx_v)
            pltpu.async_copy(h_hbm.at[idx_v], rows_v, sem).wait()
            pltpu.sync_copy(rows_v, out_hbm.at[pl.ds(off, CH)])

    return k(h, src)


def _sc_scatter(m, dst, n_nodes):
    M = m.shape[0]
    per_w = M // NW
    n_chunks = per_w // CH
    rpt = n_nodes // NS

    @functools.partial(
        pl.kernel,
        out_type=jax.ShapeDtypeStruct((NC, n_nodes, EMB), jnp.float32),
        mesh=_MESH(),
        scratch_types=[
            pltpu.VMEM((CH,), jnp.int32),
            pltpu.VMEM((CH, EMB), jnp.float32),
            pltpu.VMEM_SHARED((n_nodes, EMB), jnp.float32),
            pltpu.SemaphoreType.DMA,
        ],
        compiler_params=_SC_PARAMS,
    )
    def k(m_hbm, dst_hbm, out_hbm, idx_v, rows_v, acc_sh, sem):
        cid = lax.axis_index("c")
        sid = lax.axis_index("s")
        wid = sid * NC + cid

        def zero_row(i, _):
            rows_v[i, :] = jnp.zeros((EMB,), jnp.float32)
            return 0
        lax.fori_loop(0, rpt, zero_row, 0)
        pltpu.sync_copy(rows_v.at[pl.ds(0, rpt)], acc_sh.at[pl.ds(sid * rpt, rpt)])
        plsc.subcore_barrier()

        base = wid * per_w
        for c in range(n_chunks):
            off = base + c * CH
            pltpu.sync_copy(dst_hbm.at[pl.ds(off, CH)], idx_v)
            pltpu.sync_copy(m_hbm.at[pl.ds(off, CH)], rows_v)
            pltpu.sync_copy(rows_v, acc_sh.at[idx_v], add=True)
        plsc.subcore_barrier()

        pltpu.sync_copy(acc_sh.at[pl.ds(sid * rpt, rpt)], rows_v.at[pl.ds(0, rpt)])
        pltpu.sync_copy(rows_v.at[pl.ds(0, rpt)], out_hbm.at[cid, pl.ds(sid * rpt, rpt)])

    return k(m, dst)


def _dot(x, y):
    return jax.lax.dot_general(x, y, (((1,), (0,)), ((), ())),
                               preferred_element_type=jnp.float32)


RB = EB // 8
PK = 8 * EMB
ZW = 8 * EMB * EMB


def _msg_body(a_ref, hj_ref, r_ref, t_ref, w2_ref, bt_ref, out_ref):
    a = a_ref[...].astype(jnp.bfloat16)
    hj = hj_ref[...].astype(jnp.bfloat16)
    z = _dot(a, r_ref[...]) * _dot(hj, t_ref[...])
    m = _dot(z.astype(jnp.bfloat16), w2_ref[...]) + _dot(hj, bt_ref[...])
    out_ref[...] = m


def _messages(a_p, hj_p, W2p, Btp, Rp, Tp):
    Mr = a_p.shape[0]
    return pl.pallas_call(
        _msg_body,
        grid=(Mr // RB,),
        in_specs=[
            pl.BlockSpec((RB, PK), lambda i: (i, 0)),
            pl.BlockSpec((RB, PK), lambda i: (i, 0)),
            pl.BlockSpec((PK, ZW), lambda i: (0, 0)),
            pl.BlockSpec((PK, ZW), lambda i: (0, 0)),
            pl.BlockSpec((ZW, PK), lambda i: (0, 0)),
            pl.BlockSpec((PK, PK), lambda i: (0, 0)),
        ],
        out_specs=pl.BlockSpec((RB, PK), lambda i: (i, 0)),
        out_shape=jax.ShapeDtypeStruct((Mr, PK), jnp.float32),
    )(a_p, hj_p, Rp, Tp, W2p, Btp)


def _ln_body(h_ref, p_ref, g_ref, b_ref, out_ref):
    x = h_ref[...] + p_ref[0] + p_ref[1]
    mean = jnp.mean(x, axis=-1, keepdims=True)
    var = jnp.mean((x - mean) ** 2, axis=-1, keepdims=True)
    out_ref[...] = (x - mean) * jax.lax.rsqrt(var + 1e-5) * g_ref[...] + b_ref[...]


def _layer_norm(h, partials, gamma, beta):
    N = h.shape[0]
    NB = 2000
    return pl.pallas_call(
        _ln_body,
        grid=(N // NB,),
        in_specs=[
            pl.BlockSpec((NB, EMB), lambda i: (i, 0)),
            pl.BlockSpec((NC, NB, EMB), lambda i: (0, i, 0)),
            pl.BlockSpec((1, EMB), lambda i: (0, 0)),
            pl.BlockSpec((1, EMB), lambda i: (0, 0)),
        ],
        out_specs=pl.BlockSpec((NB, EMB), lambda i: (i, 0)),
        out_shape=jax.ShapeDtypeStruct((N, EMB), jnp.float32),
    )(h, partials, gamma.reshape(1, EMB), beta.reshape(1, EMB))


def kernel(h, edge_index, edge_attr, W_lin, b_lin, ln_gamma, ln_beta):
    dst = edge_index[0].astype(jnp.int32)
    src = edge_index[1].astype(jnp.int32)
    M = edge_attr.shape[0]
    W2 = W_lin.reshape(EMB, EMB, EMB).transpose(2, 1, 0).reshape(EMB * EMB, EMB)
    Bt = b_lin.reshape(EMB, EMB).T
    eye = jnp.eye(EMB, dtype=jnp.float32)
    R = jnp.repeat(eye, EMB, axis=1)
    T = jnp.tile(eye, (1, EMB))
    eye8 = jnp.eye(8, dtype=jnp.float32)
    Rp = jnp.kron(eye8, R).astype(jnp.bfloat16)
    Tp = jnp.kron(eye8, T).astype(jnp.bfloat16)
    W2p = jnp.kron(eye8, W2).astype(jnp.bfloat16)
    Btp = jnp.kron(eye8, Bt).astype(jnp.bfloat16)

    h_j = _sc_gather(h, src)
    m = _messages(edge_attr.reshape(M // 8, PK), h_j.reshape(M // 8, PK),
                  W2p, Btp, Rp, Tp)
    partials = _sc_scatter(m.reshape(M, EMB), dst, h.shape[0])
    return _layer_norm(h, partials, ln_gamma, ln_beta)


# speedup vs baseline: 1.0532x; 1.0001x over previous
import functools
import jax
import jax.numpy as jnp
from jax import lax
from jax.experimental import layout as jex_layout
from jax.experimental import pallas as pl
from jax.experimental.pallas import tpu as pltpu
from jax.experimental.pallas import tpu_sc as plsc

EMB = 16
NC = 2
NS = 16
NW = NC * NS
EB = 16000
CH = 2000

_MESH = functools.partial(
    plsc.VectorSubcoreMesh, core_axis_name="c", subcore_axis_name="s",
    num_cores=NC, num_subcores=NS)
_SC_PARAMS = pltpu.CompilerParams(use_tc_tiling_on_sc=False)


def _sc_gather(h, src):
    M = src.shape[0]
    per_w = M // NW
    n_chunks = per_w // CH

    @functools.partial(
        pl.kernel,
        out_type=jax.ShapeDtypeStruct((M, EMB), jnp.float32),
        mesh=_MESH(),
        scratch_types=[
            pltpu.VMEM((CH,), jnp.int32),
            pltpu.VMEM((CH, EMB), jnp.float32),
            pltpu.SemaphoreType.DMA,
        ],
        compiler_params=_SC_PARAMS,
    )
    def k(h_hbm, src_hbm, out_hbm, idx_v, rows_v, sem):
        wid = lax.axis_index("s") * NC + lax.axis_index("c")
        base = wid * per_w
        for c in range(n_chunks):
            off = base + c * CH
            pltpu.sync_copy(src_hbm.at[pl.ds(off, CH)], idx_v)
            pltpu.async_copy(h_hbm.at[idx_v], rows_v, sem).wait()
            pltpu.sync_copy(rows_v, out_hbm.at[pl.ds(off, CH)])

    return k(h, src)


def _sc_scatter(m, dst, n_nodes):
    M = m.shape[0]
    per_w = M // NW
    n_chunks = per_w // CH
    rpt = n_nodes // NS

    @functools.partial(
        pl.kernel,
        out_type=jax.ShapeDtypeStruct((NC, n_nodes, EMB), jnp.float32),
        mesh=_MESH(),
        scratch_types=[
            pltpu.VMEM((CH,), jnp.int32),
            pltpu.VMEM((CH, EMB), jnp.float32),
            pltpu.VMEM_SHARED((n_nodes, EMB), jnp.float32),
            pltpu.SemaphoreType.DMA,
        ],
        compiler_params=_SC_PARAMS,
    )
    def k(m_hbm, dst_hbm, out_hbm, idx_v, rows_v, acc_sh, sem):
        cid = lax.axis_index("c")
        sid = lax.axis_index("s")
        wid = sid * NC + cid

        def zero_row(i, _):
            rows_v[i, :] = jnp.zeros((EMB,), jnp.float32)
            return 0
        lax.fori_loop(0, rpt, zero_row, 0)
        pltpu.sync_copy(rows_v.at[pl.ds(0, rpt)], acc_sh.at[pl.ds(sid * rpt, rpt)])
        plsc.subcore_barrier()

        base = wid * per_w
        for c in range(n_chunks):
            off = base + c * CH
            pltpu.sync_copy(dst_hbm.at[pl.ds(off, CH)], idx_v)
            pltpu.sync_copy(m_hbm.at[pl.ds(off, CH)], rows_v)
            pltpu.sync_copy(rows_v, acc_sh.at[idx_v], add=True)
        plsc.subcore_barrier()

        pltpu.sync_copy(acc_sh.at[pl.ds(sid * rpt, rpt)], rows_v.at[pl.ds(0, rpt)])
        pltpu.sync_copy(rows_v.at[pl.ds(0, rpt)], out_hbm.at[cid, pl.ds(sid * rpt, rpt)])

    return k(m, dst)


def _dot(x, y):
    return jax.lax.dot_general(x, y, (((1,), (0,)), ((), ())),
                               preferred_element_type=jnp.float32)


RB = EB // 8
PK = 8 * EMB
ZW = 8 * EMB * EMB


def _msg_body(a_ref, hj_ref, r_ref, t_ref, w2_ref, bt_ref, out_ref):
    a = a_ref[...].astype(jnp.bfloat16)
    hj = hj_ref[...].astype(jnp.bfloat16)
    z = _dot(a, r_ref[...]) * _dot(hj, t_ref[...])
    m = _dot(z.astype(jnp.bfloat16), w2_ref[...]) + _dot(hj, bt_ref[...])
    out_ref[...] = m


def _messages(a_p, hj_p, W2p, Btp, Rp, Tp):
    Mr = a_p.shape[0]
    return pl.pallas_call(
        _msg_body,
        grid=(Mr // RB,),
        in_specs=[
            pl.BlockSpec((RB, PK), lambda i: (i, 0)),
            pl.BlockSpec((RB, PK), lambda i: (i, 0)),
            pl.BlockSpec((PK, ZW), lambda i: (0, 0)),
            pl.BlockSpec((PK, ZW), lambda i: (0, 0)),
            pl.BlockSpec((ZW, PK), lambda i: (0, 0)),
            pl.BlockSpec((PK, PK), lambda i: (0, 0)),
        ],
        out_specs=pl.BlockSpec((RB, PK), lambda i: (i, 0)),
        out_shape=jax.ShapeDtypeStruct((Mr, PK), jnp.float32),
    )(a_p, hj_p, Rp, Tp, W2p, Btp)


def _ln_body(h_ref, p_ref, g_ref, b_ref, out_ref):
    x = h_ref[...] + p_ref[0] + p_ref[1]
    mean = jnp.mean(x, axis=-1, keepdims=True)
    var = jnp.mean((x - mean) ** 2, axis=-1, keepdims=True)
    out_ref[...] = (x - mean) * jax.lax.rsqrt(var + 1e-5) * g_ref[...] + b_ref[...]


def _layer_norm(h, partials, gamma, beta):
    N = h.shape[0]
    NB = 2000
    return pl.pallas_call(
        _ln_body,
        grid=(N // NB,),
        in_specs=[
            pl.BlockSpec((NB, EMB), lambda i: (i, 0)),
            pl.BlockSpec((NC, NB, EMB), lambda i: (0, i, 0)),
            pl.BlockSpec((1, EMB), lambda i: (0, 0)),
            pl.BlockSpec((1, EMB), lambda i: (0, 0)),
        ],
        out_specs=pl.BlockSpec((NB, EMB), lambda i: (i, 0)),
        out_shape=jax.ShapeDtypeStruct((N, EMB), jnp.float32),
    )(h, partials, gamma.reshape(1, EMB), beta.reshape(1, EMB))


def kernel(h, edge_index, edge_attr, W_lin, b_lin, ln_gamma, ln_beta):
    dst = edge_index[0].astype(jnp.int32)
    src = edge_index[1].astype(jnp.int32)
    M = edge_attr.shape[0]
    W2 = W_lin.reshape(EMB, EMB, EMB).transpose(2, 1, 0).reshape(EMB * EMB, EMB)
    Bt = b_lin.reshape(EMB, EMB).T
    eye = jnp.eye(EMB, dtype=jnp.float32)
    R = jnp.repeat(eye, EMB, axis=1)
    T = jnp.tile(eye, (1, EMB))
    eye8 = jnp.eye(8, dtype=jnp.float32)
    Rp = jnp.kron(eye8, R).astype(jnp.bfloat16)
    Tp = jnp.kron(eye8, T).astype(jnp.bfloat16)
    W2p = jnp.kron(eye8, W2).astype(jnp.bfloat16)
    Btp = jnp.kron(eye8, Bt).astype(jnp.bfloat16)

    edge_attr = jex_layout.with_layout_constraint(
        edge_attr, jex_layout.Layout((0, 1)))

    h_j = _sc_gather(h, src)
    m = _messages(edge_attr.reshape(M // 8, PK), h_j.reshape(M // 8, PK),
                  W2p, Btp, Rp, Tp)
    partials = _sc_scatter(m.reshape(M, EMB), dst, h.shape[0])
    return _layer_norm(h, partials, ln_gamma, ln_beta)
